# baseline (speedup 1.0000x reference)
import functools

import jax
import jax.numpy as jnp
from jax import lax
from jax.experimental import pallas as pl
from jax.experimental.pallas import tpu as pltpu
from jax.experimental.pallas import tpu_sc as plsc


@functools.cache
def _build(E, N, D):
    info = plsc.get_sparse_core_info()
    NW = info.num_cores * info.num_subcores
    L = info.num_lanes
    assert E % NW == 0 and D % (2 * L) == 0
    W = D // 2
    EPW = E // NW
    B = 80
    assert EPW % B == 0 and B % L == 0
    NCHUNK = EPW // B

    mesh = plsc.VectorSubcoreMesh(core_axis_name="c", subcore_axis_name="s")

    @functools.partial(
        pl.kernel,
        mesh=mesh,
        compiler_params=pltpu.CompilerParams(
            needs_layout_passes=False, use_tc_tiling_on_sc=False),
        out_type=jax.ShapeDtypeStruct((E,), jnp.float32),
        scratch_types=[
            pltpu.VMEM((EPW,), jnp.int32),
            pltpu.VMEM((EPW,), jnp.int32),
            pltpu.VMEM((B, W), jnp.int32),
            pltpu.VMEM((B, W), jnp.int32),
            pltpu.VMEM((B, W), jnp.int32),
            pltpu.VMEM((B, W), jnp.int32),
            pltpu.VMEM((B,), jnp.float32),
            pltpu.VMEM_SHARED((N, W), jnp.int32),
            pltpu.SemaphoreType.DMA,
            pltpu.SemaphoreType.DMA,
        ],
    )
    def scores_kernel(h_hbm, src_hbm, dst_hbm, out_hbm,
                      src_v, dst_v, srow0, drow0, srow1, drow1, outc,
                      h_sp, sem0, sem1):
        sid = lax.axis_index("s")
        wid = sid * info.num_cores + lax.axis_index("c")
        base = wid * EPW
        per, extra = divmod(N, info.num_subcores)
        off = 0
        for i in range(info.num_subcores):
            rows = per + (1 if i < extra else 0)

            @pl.when(sid == i)
            def _stage(off=off, rows=rows):
                pltpu.sync_copy(h_hbm.at[pl.ds(off, rows)],
                                h_sp.at[pl.ds(off, rows)])

            off += rows
        pltpu.sync_copy(src_hbm.at[pl.ds(base, EPW)], src_v)
        pltpu.sync_copy(dst_hbm.at[pl.ds(base, EPW)], dst_v)
        plsc.subcore_barrier()

        bufs = ((srow0, drow0, sem0), (srow1, drow1, sem1))

        def issue(off, b):
            srow, drow, sem = bufs[b]
            pltpu.async_copy(h_sp.at[src_v.at[pl.ds(off, B)]], srow, sem)
            pltpu.async_copy(h_sp.at[dst_v.at[pl.ds(off, B)]], drow, sem)

        def wait(b):
            srow, drow, sem = bufs[b]
            pltpu.make_async_copy(h_hbm.at[pl.ds(0, B)], srow, sem).wait()
            pltpu.make_async_copy(h_hbm.at[pl.ds(0, B)], drow, sem).wait()

        def compute(off, b):
            srow, drow, _ = bufs[b]
            lanes = lax.iota(jnp.int32, L)
            for g in range(B // L):
                parts = []
                for i in range(L):
                    e = g * L + i
                    acc = None
                    for k in range(W // L):
                        sv = plsc.bitcast(srow[e, pl.ds(k * L, L)], jnp.bfloat16)
                        dv = plsc.bitcast(drow[e, pl.ds(k * L, L)], jnp.bfloat16)
                        sa, sb = plsc.unpack(sv, format=plsc.PackFormat.INTERLEAVED)
                        da, db = plsc.unpack(dv, format=plsc.PackFormat.INTERLEAVED)
                        term = sa * da + sb * db
                        acc = term if acc is None else acc + term
                    for s in (8, 4, 2, 1):
                        perm = lax.gather(
                            acc, (lanes ^ s)[:, None],
                            lax.GatherDimensionNumbers(
                                offset_dims=(),
                                collapsed_slice_dims=(0,),
                                start_index_map=(0,)),
                            (1,),
                            mode=lax.GatherScatterMode.PROMISE_IN_BOUNDS)
                        acc = acc + perm
                    parts.append(jnp.where(lanes == i, acc, 0.0))
                while len(parts) > 1:
                    parts = [a + b for a, b in zip(parts[::2], parts[1::2])]
                outc[pl.ds(g * L, L)] = parts[0]
            pltpu.sync_copy(outc, out_hbm.at[pl.ds(base + off, B)])

        issue(0, 0)

        def pair(jj, carry):
            j0 = jj * 2
            issue((j0 + 1) * B, 1)
            wait(0)
            compute(j0 * B, 0)
            issue((j0 + 2) * B, 0)
            wait(1)
            compute((j0 + 1) * B, 1)
            return carry

        if NCHUNK % 2:
            lax.fori_loop(0, NCHUNK // 2, pair, 0)
            wait(0)
            compute((NCHUNK - 1) * B, 0)
        else:
            lax.fori_loop(0, NCHUNK // 2 - 1, pair, 0)
            j0 = NCHUNK - 2
            issue((j0 + 1) * B, 1)
            wait(0)
            compute(j0 * B, 0)
            wait(1)
            compute((j0 + 1) * B, 1)

    return scores_kernel


def kernel(h, edge_index):
    src = edge_index[0].astype(jnp.int32)
    dst = edge_index[1].astype(jnp.int32)
    E = src.shape[0]
    N, D = h.shape
    h_packed = lax.bitcast_convert_type(
        h.astype(jnp.bfloat16).reshape(N, D // 2, 2), jnp.int32)
    score = _build(E, N, D)(h_packed, src, dst)
    return score[:, None]


# speedup vs baseline: 1.2206x; 1.2206x over previous
import functools

import jax
import jax.numpy as jnp
from jax import lax
from jax.experimental import pallas as pl
from jax.experimental.pallas import tpu as pltpu
from jax.experimental.pallas import tpu_sc as plsc


@functools.cache
def _build(E, N, D):
    info = plsc.get_sparse_core_info()
    NW = info.num_cores * info.num_subcores
    L = info.num_lanes
    assert E % NW == 0 and D % (2 * L) == 0
    W = D // 2
    EPW = E // NW
    B = 80
    assert EPW % B == 0 and B % L == 0
    NCHUNK = EPW // B

    mesh = plsc.VectorSubcoreMesh(core_axis_name="c", subcore_axis_name="s")

    @functools.partial(
        pl.kernel,
        mesh=mesh,
        compiler_params=pltpu.CompilerParams(
            needs_layout_passes=False, use_tc_tiling_on_sc=False),
        out_type=jax.ShapeDtypeStruct((E,), jnp.float32),
        scratch_types=[
            pltpu.VMEM((EPW,), jnp.int32),
            pltpu.VMEM((EPW,), jnp.int32),
            pltpu.VMEM((B, W), jnp.int32),
            pltpu.VMEM((B, W), jnp.int32),
            pltpu.VMEM((B, W), jnp.int32),
            pltpu.VMEM((B, W), jnp.int32),
            pltpu.VMEM((B,), jnp.float32),
            pltpu.VMEM_SHARED((N, W), jnp.int32),
            pltpu.SemaphoreType.DMA,
            pltpu.SemaphoreType.DMA,
        ],
    )
    def scores_kernel(h_hbm, src_hbm, dst_hbm, out_hbm,
                      src_v, dst_v, srow0, drow0, srow1, drow1, outc,
                      h_sp, sem0, sem1):
        sid = lax.axis_index("s")
        wid = sid * info.num_cores + lax.axis_index("c")
        base = wid * EPW
        per, extra = divmod(N, info.num_subcores)
        off = 0
        for i in range(info.num_subcores):
            rows = per + (1 if i < extra else 0)

            @pl.when(sid == i)
            def _stage(off=off, rows=rows):
                pltpu.sync_copy(h_hbm.at[pl.ds(off, rows)],
                                h_sp.at[pl.ds(off, rows)])

            off += rows
        pltpu.sync_copy(src_hbm.at[pl.ds(base, EPW)], src_v)
        pltpu.sync_copy(dst_hbm.at[pl.ds(base, EPW)], dst_v)
        plsc.subcore_barrier()

        bufs = ((srow0, drow0, sem0), (srow1, drow1, sem1))

        def issue(off, b):
            srow, drow, sem = bufs[b]
            pltpu.async_copy(h_sp.at[src_v.at[pl.ds(off, B)]], srow, sem)
            pltpu.async_copy(h_sp.at[dst_v.at[pl.ds(off, B)]], drow, sem)

        def wait(b):
            srow, drow, sem = bufs[b]
            pltpu.make_async_copy(h_hbm.at[pl.ds(0, B)], srow, sem).wait()
            pltpu.make_async_copy(h_hbm.at[pl.ds(0, B)], drow, sem).wait()

        def compute(off, b):
            srow, drow, _ = bufs[b]
            lanes = lax.iota(jnp.int32, L)
            for g in range(B // L):
                tot = jnp.zeros((L,), jnp.float32)
                for i in range(L):
                    e = g * L + i
                    acc = None
                    for k in range(W // L):
                        sv = plsc.bitcast(srow[e, pl.ds(k * L, L)], jnp.bfloat16)
                        dv = plsc.bitcast(drow[e, pl.ds(k * L, L)], jnp.bfloat16)
                        sa, sb = plsc.unpack(sv, format=plsc.PackFormat.INTERLEAVED)
                        da, db = plsc.unpack(dv, format=plsc.PackFormat.INTERLEAVED)
                        term = sa * da + sb * db
                        acc = term if acc is None else acc + term
                    tot = jnp.where(lanes == i, jnp.sum(acc), tot)
                outc[pl.ds(g * L, L)] = tot
            pltpu.sync_copy(outc, out_hbm.at[pl.ds(base + off, B)])

        issue(0, 0)

        def pair(jj, carry):
            j0 = jj * 2
            issue((j0 + 1) * B, 1)
            wait(0)
            compute(j0 * B, 0)
            issue((j0 + 2) * B, 0)
            wait(1)
            compute((j0 + 1) * B, 1)
            return carry

        if NCHUNK % 2:
            lax.fori_loop(0, NCHUNK // 2, pair, 0)
            wait(0)
            compute((NCHUNK - 1) * B, 0)
        else:
            lax.fori_loop(0, NCHUNK // 2 - 1, pair, 0)
            j0 = NCHUNK - 2
            issue((j0 + 1) * B, 1)
            wait(0)
            compute(j0 * B, 0)
            wait(1)
            compute((j0 + 1) * B, 1)

    return scores_kernel


def kernel(h, edge_index):
    src = edge_index[0].astype(jnp.int32)
    dst = edge_index[1].astype(jnp.int32)
    E = src.shape[0]
    N, D = h.shape
    h_packed = lax.bitcast_convert_type(
        h.astype(jnp.bfloat16).reshape(N, D // 2, 2), jnp.int32)
    score = _build(E, N, D)(h_packed, src, dst)
    return score[:, None]
